# baseline (speedup 1.0000x reference)
import functools

import jax
import jax.numpy as jnp
from jax import lax
from jax.experimental import pallas as pl
from jax.experimental.pallas import tpu as pltpu
from jax.experimental.pallas import tpu_sc as plsc

N = 10000
D = 512
K2 = 8
NC, NS = 2, 16
NW = NC * NS
BP = 10240
NPW = BP // NW
NG = NPW // 16


def _lrelu(x):
    return jnp.where(x >= 0, x, x * 0.2)


def _proj_scal_body(x_ref, w_ref, o_ref):
    o_ref[...] = jnp.dot(x_ref[...], w_ref[...],
                         preferred_element_type=jnp.float32)


def _proj_scalars(comb, wc128):
    grid = 10
    blk = (2 * N) // grid
    return pl.pallas_call(
        _proj_scal_body,
        grid=(grid,),
        in_specs=[
            pl.BlockSpec((blk, D), lambda i: (i, 0)),
            pl.BlockSpec((D, 128), lambda i: (0, 0)),
        ],
        out_specs=pl.BlockSpec((blk, 128), lambda i: (i, 0)),
        out_shape=jax.ShapeDtypeStruct((2 * N, 128), jnp.float32),
    )(comb, wc128)


def _out_proj_body(x_ref, w_ref, b_ref, o_ref):
    acc = lax.dot_general(x_ref[...], w_ref[...],
                          (((1,), (1,)), ((), ())),
                          preferred_element_type=jnp.float32)
    o_ref[...] = acc + b_ref[...]


def _out_proj(x, w, b):
    grid = BP // 512
    return pl.pallas_call(
        _out_proj_body,
        grid=(grid,),
        in_specs=[
            pl.BlockSpec((512, D), lambda i: (i, 0)),
            pl.BlockSpec((D, D), lambda i: (0, 0)),
            pl.BlockSpec((1, D), lambda i: (0, 0)),
        ],
        out_specs=pl.BlockSpec((512, D), lambda i: (i, 0)),
        out_shape=jax.ShapeDtypeStruct((BP, D), jnp.float32),
    )(x, w, b)


def _sc_body(comb_hbm, na_hbm, nb_hbm, tgt_hbm,
             sa_h, qa_h, sb_h, qb_h, wt_h, qt_h,
             out_hbm,
             tsa, tqa, tsb, tqb, twt, tqt,
             nav, nbv, tgtv, betv,
             rowsb, idxb, outb, rsem):
    wid = lax.axis_index("s") * NC + lax.axis_index("c")
    base = wid * NPW

    pltpu.sync_copy(sa_h, tsa)
    pltpu.sync_copy(qa_h, tqa)
    pltpu.sync_copy(sb_h, tsb)
    pltpu.sync_copy(qb_h, tqb)
    pltpu.sync_copy(wt_h, twt)
    pltpu.sync_copy(qt_h, tqt)
    pltpu.sync_copy(na_hbm.at[pl.ds(base * K2, NPW * K2)], nav)
    pltpu.sync_copy(nb_hbm.at[pl.ds(base * K2, NPW * K2)], nbv)
    pltpu.sync_copy(tgt_hbm.at[pl.ds(base, NPW)], tgtv)

    iota = lax.broadcasted_iota(jnp.int32, (16,), 0)

    def group_a(g, carry):
        gb = g * 16
        tgt = tgtv[pl.ds(gb, 16)]
        t_w = plsc.load_gather(twt, [tgt])
        t_q = plsc.load_gather(tqt, [tgt])
        qacc_a = jnp.zeros((16,), jnp.float32)
        qacc_b = jnp.zeros((16,), jnp.float32)
        ek = []
        for k in range(K2):
            ids = plsc.load_gather(nav, [iota * K2 + (gb * K2 + k)])
            qacc_a = qacc_a + plsc.load_gather(tqa, [ids])
            s = plsc.load_gather(tsa, [ids])
            ek.append(jnp.exp(_lrelu(t_w + s)))
        for k in range(K2):
            ids = plsc.load_gather(nbv, [iota * K2 + (gb * K2 + k)])
            qacc_b = qacc_b + plsc.load_gather(tqb, [ids])
            s = plsc.load_gather(tsb, [ids])
            ek.append(jnp.exp(_lrelu(t_w + s)))
        log_a = _lrelu(t_q + qacc_a * (1.0 / K2))
        log_b = _lrelu(t_q + qacc_b * (1.0 / K2))
        m = jnp.maximum(log_a, log_b)
        ea = jnp.exp(log_a - m)
        eb = jnp.exp(log_b - m)
        inv = 1.0 / (ea + eb)
        al_a = ea * inv
        al_b = eb * inv
        u = [ek[k] * al_a for k in range(K2)] + \
            [ek[K2 + k] * al_b for k in range(K2)]
        mu = u[0]
        for k in range(1, 16):
            mu = jnp.maximum(mu, u[k])
        w = [jnp.exp(u[k] - mu) for k in range(16)]
        ssum = w[0]
        for k in range(1, 16):
            ssum = ssum + w[k]
        inv_s = 1.0 / ssum
        for k in range(16):
            plsc.store_scatter(betv, [iota * 16 + (gb * 16 + k)],
                               w[k] * inv_s)
        return carry

    lax.fori_loop(0, NG, group_a, 0)

    def _stage_pair_idx(p, slot):
        n0 = 2 * p
        ia = plsc.load_gather(nav, [n0 * K2 + (iota & (K2 - 1))])
        ib = plsc.load_gather(nbv, [n0 * K2 + (iota & (K2 - 1))]) + N
        idxb[pl.ds(slot * 48, 16)] = jnp.where(iota < K2, ia, ib)
        ia = plsc.load_gather(nav, [(n0 + 1) * K2 + (iota & (K2 - 1))])
        ib = plsc.load_gather(nbv, [(n0 + 1) * K2 + (iota & (K2 - 1))]) + N
        idxb[pl.ds(slot * 48 + 16, 16)] = jnp.where(iota < K2, ia, ib)
        tv = plsc.load_gather(tgtv, [jnp.minimum(n0 + iota, NPW - 1)])
        idxb[pl.ds(slot * 48 + 32, 16)] = tv

    def _fire_pair(slot):
        return pltpu.async_copy(
            comb_hbm.at[idxb.at[pl.ds(slot * 48, 40)]],
            rowsb.at[pl.ds(slot * 40, 40)], rsem)

    def _compute_node(n, rowbase, tgtrow, orow):
        for c in range(D // 16):
            outb[orow, pl.ds(c * 16, 16)] = rowsb[tgtrow, pl.ds(c * 16, 16)]
        for k in range(16):
            bk = plsc.load_gather(
                betv, [jnp.full((16,), n * 16 + k, jnp.int32)])
            for c in range(D // 16):
                plsc.addupdate(outb.at[orow, pl.ds(c * 16, 16)],
                               bk * rowsb[rowbase + k, pl.ds(c * 16, 16)])

    def _compute_pair(p, slot):
        rb = slot * 40
        _compute_node(2 * p, rb, rb + 32, (p & 7) * 2)
        _compute_node(2 * p + 1, rb + 16, rb + 33, (p & 7) * 2 + 1)

    NPAIR = NPW // 2
    _stage_pair_idx(0, 0)
    _fire_pair(0).wait()

    def pair2_b(q, carry):
        p0 = 2 * q
        p1 = 2 * q + 1
        _stage_pair_idx(p1, 1)
        d1 = _fire_pair(1)
        _compute_pair(p0, 0)
        d1.wait()

        pn = jnp.minimum(p1 + 1, NPAIR - 1)
        _stage_pair_idx(pn, 0)
        d0 = _fire_pair(0)
        _compute_pair(p1, 1)

        @pl.when((q & 3) == 3)
        def _():
            pltpu.sync_copy(outb,
                            out_hbm.at[pl.ds(base + (q // 4) * 16, 16)])

        d0.wait()
        return carry

    lax.fori_loop(0, NPAIR // 2, pair2_b, 0)


def _sc_attention(comb, na_p, nb_p, tgt_p, sa, qa, sb, qb, wt, qt):
    mesh = plsc.VectorSubcoreMesh(core_axis_name="c", subcore_axis_name="s",
                                  num_cores=NC, num_subcores=NS)
    f32, i32 = jnp.float32, jnp.int32
    kern = functools.partial(
        pl.kernel,
        out_type=jax.ShapeDtypeStruct((BP, D), f32),
        mesh=mesh,
        compiler_params=pltpu.CompilerParams(needs_layout_passes=False),
        scratch_types=[
            pltpu.VMEM((N,), f32), pltpu.VMEM((N,), f32),
            pltpu.VMEM((N,), f32), pltpu.VMEM((N,), f32),
            pltpu.VMEM((N,), f32), pltpu.VMEM((N,), f32),
            pltpu.VMEM((NPW * K2,), i32), pltpu.VMEM((NPW * K2,), i32),
            pltpu.VMEM((NPW,), i32),
            pltpu.VMEM((NPW * 16,), f32),
            pltpu.VMEM((80, D), f32),
            pltpu.VMEM((96,), i32),
            pltpu.VMEM((16, D), f32),
            pltpu.SemaphoreType.DMA,
        ],
    )(_sc_body)
    return kern(comb, na_p, nb_p, tgt_p, sa, qa, sb, qb, wt, qt)


def kernel(target_ids, feats_A, feats_B, neigh_ids_A, neigh_ids_B,
           type_attn_query, node_attn_w, proj_w, proj_b):
    i32 = jnp.int32
    comb = jnp.concatenate([feats_A, feats_B], axis=0)

    q = type_attn_query[0]
    w = node_attn_w[0]
    wc = jnp.stack([w[D:], q[D:], w[:D], q[:D]], axis=1)
    wc128 = jnp.pad(wc, ((0, 0), (0, 124)))

    scal = _proj_scalars(comb, wc128)
    sa, qa = scal[:N, 0], scal[:N, 1]
    wt, qt = scal[:N, 2], scal[:N, 3]
    sb, qb = scal[N:, 0], scal[N:, 1]

    pad = BP - N
    tgt_p = jnp.pad(target_ids.astype(i32), (0, pad))
    na_p = jnp.pad(neigh_ids_A.astype(i32), ((0, pad), (0, 0))).reshape(-1)
    nb_p = jnp.pad(neigh_ids_B.astype(i32), ((0, pad), (0, 0))).reshape(-1)

    out_pre = _sc_attention(comb, na_p, nb_p, tgt_p,
                            sa, qa, sb, qb, wt, qt)

    y = _out_proj(out_pre, proj_w, proj_b.reshape(1, D))
    return y[:N]


# speedup vs baseline: 2.2899x; 2.2899x over previous
import functools

import jax
import jax.numpy as jnp
from jax import lax
from jax.experimental import pallas as pl
from jax.experimental.pallas import tpu as pltpu
from jax.experimental.pallas import tpu_sc as plsc

N = 10000
D = 512
K2 = 8
NC, NS = 2, 16
NW = NC * NS
BP = 10240
NPW = BP // NW
NG = NPW // 16


def _lrelu(x):
    return jnp.where(x >= 0, x, x * 0.2)


def _proj_scal_body(x_ref, w_ref, o_ref):
    o_ref[...] = jnp.dot(x_ref[...], w_ref[...],
                         preferred_element_type=jnp.float32)


def _proj_scalars(comb, wc128):
    grid = 10
    blk = (2 * N) // grid
    return pl.pallas_call(
        _proj_scal_body,
        grid=(grid,),
        in_specs=[
            pl.BlockSpec((blk, D), lambda i: (i, 0)),
            pl.BlockSpec((D, 128), lambda i: (0, 0)),
        ],
        out_specs=pl.BlockSpec((blk, 128), lambda i: (i, 0)),
        out_shape=jax.ShapeDtypeStruct((2 * N, 128), jnp.float32),
    )(comb, wc128)


def _out_proj_body(x_ref, w_ref, b_ref, o_ref):
    acc = lax.dot_general(x_ref[...], w_ref[...],
                          (((1,), (1,)), ((), ())),
                          preferred_element_type=jnp.float32)
    o_ref[...] = acc + b_ref[...]


def _out_proj(x, w, b):
    grid = BP // 512
    return pl.pallas_call(
        _out_proj_body,
        grid=(grid,),
        in_specs=[
            pl.BlockSpec((512, D), lambda i: (i, 0)),
            pl.BlockSpec((D, D), lambda i: (0, 0)),
            pl.BlockSpec((1, D), lambda i: (0, 0)),
        ],
        out_specs=pl.BlockSpec((512, D), lambda i: (i, 0)),
        out_shape=jax.ShapeDtypeStruct((BP, D), jnp.float32),
    )(x, w, b)


def _sc_body(comb_hbm, na_hbm, nb_hbm, tgt_hbm,
             sa_h, qa_h, sb_h, qb_h, wt_h, qt_h,
             out_hbm,
             tsa, tqa, tsb, tqb, twt, tqt,
             nav, nbv, tgtv, betv,
             rowsb, idxb, outb, rsem):
    wid = lax.axis_index("s") * NC + lax.axis_index("c")
    base = wid * NPW

    pltpu.sync_copy(sa_h, tsa)
    pltpu.sync_copy(qa_h, tqa)
    pltpu.sync_copy(sb_h, tsb)
    pltpu.sync_copy(qb_h, tqb)
    pltpu.sync_copy(wt_h, twt)
    pltpu.sync_copy(qt_h, tqt)
    pltpu.sync_copy(na_hbm.at[pl.ds(base * K2, NPW * K2)], nav)
    pltpu.sync_copy(nb_hbm.at[pl.ds(base * K2, NPW * K2)], nbv)
    pltpu.sync_copy(tgt_hbm.at[pl.ds(base, NPW)], tgtv)

    iota = lax.broadcasted_iota(jnp.int32, (16,), 0)

    def group_a(g, carry):
        gb = g * 16
        tgt = tgtv[pl.ds(gb, 16)]
        t_w = plsc.load_gather(twt, [tgt])
        t_q = plsc.load_gather(tqt, [tgt])
        qacc_a = jnp.zeros((16,), jnp.float32)
        qacc_b = jnp.zeros((16,), jnp.float32)
        ek = []
        for k in range(K2):
            ids = plsc.load_gather(nav, [iota * K2 + (gb * K2 + k)])
            qacc_a = qacc_a + plsc.load_gather(tqa, [ids])
            s = plsc.load_gather(tsa, [ids])
            ek.append(jnp.exp(_lrelu(t_w + s)))
        for k in range(K2):
            ids = plsc.load_gather(nbv, [iota * K2 + (gb * K2 + k)])
            qacc_b = qacc_b + plsc.load_gather(tqb, [ids])
            s = plsc.load_gather(tsb, [ids])
            ek.append(jnp.exp(_lrelu(t_w + s)))
        log_a = _lrelu(t_q + qacc_a * (1.0 / K2))
        log_b = _lrelu(t_q + qacc_b * (1.0 / K2))
        m = jnp.maximum(log_a, log_b)
        ea = jnp.exp(log_a - m)
        eb = jnp.exp(log_b - m)
        inv = 1.0 / (ea + eb)
        al_a = ea * inv
        al_b = eb * inv
        u = [ek[k] * al_a for k in range(K2)] + \
            [ek[K2 + k] * al_b for k in range(K2)]
        mu = u[0]
        for k in range(1, 16):
            mu = jnp.maximum(mu, u[k])
        w = [jnp.exp(u[k] - mu) for k in range(16)]
        ssum = w[0]
        for k in range(1, 16):
            ssum = ssum + w[k]
        inv_s = 1.0 / ssum
        for k in range(16):
            plsc.store_scatter(betv, [iota * 16 + (gb * 16 + k)],
                               w[k] * inv_s)
        return carry

    lax.fori_loop(0, NG, group_a, 0)

    def _stage_pair_idx(p, slot):
        n0 = 2 * p
        ia = plsc.load_gather(nav, [n0 * K2 + (iota & (K2 - 1))])
        ib = plsc.load_gather(nbv, [n0 * K2 + (iota & (K2 - 1))]) + N
        idxb[pl.ds(slot * 48, 16)] = jnp.where(iota < K2, ia, ib)
        ia = plsc.load_gather(nav, [(n0 + 1) * K2 + (iota & (K2 - 1))])
        ib = plsc.load_gather(nbv, [(n0 + 1) * K2 + (iota & (K2 - 1))]) + N
        idxb[pl.ds(slot * 48 + 16, 16)] = jnp.where(iota < K2, ia, ib)
        tv = plsc.load_gather(tgtv, [jnp.minimum(n0 + iota, NPW - 1)])
        idxb[pl.ds(slot * 48 + 32, 16)] = tv

    def _fire_pair(slot):
        return pltpu.async_copy(
            comb_hbm.at[idxb.at[pl.ds(slot * 48, 40)]],
            rowsb.at[pl.ds(slot * 40, 40)], rsem)

    def _compute_node(n, rowbase, tgtrow, orow):
        for h in range(2):
            c0 = h * (D // 32)
            accs = [rowsb[tgtrow, pl.ds((c0 + c) * 16, 16)]
                    for c in range(D // 32)]
            for k in range(16):
                bk = plsc.load_gather(
                    betv, [jnp.full((16,), n * 16 + k, jnp.int32)])
                for c in range(D // 32):
                    accs[c] = accs[c] + bk * rowsb[rowbase + k,
                                                   pl.ds((c0 + c) * 16, 16)]
            for c in range(D // 32):
                outb[orow, pl.ds((c0 + c) * 16, 16)] = accs[c]

    NPAIR = NPW // 2
    _stage_pair_idx(0, 0)
    _fire_pair(0).wait()

    def pair_b(p, carry):
        slot = p & 1
        nslot = (p + 1) & 1
        pn = jnp.minimum(p + 1, NPAIR - 1)
        _stage_pair_idx(pn, nslot)
        rdesc = _fire_pair(nslot)

        rb = slot * 40
        _compute_node(2 * p, rb, rb + 32, (p & 7) * 2)
        _compute_node(2 * p + 1, rb + 16, rb + 33, (p & 7) * 2 + 1)

        @pl.when((p & 7) == 7)
        def _():
            pltpu.sync_copy(outb,
                            out_hbm.at[pl.ds(base + (p // 8) * 16, 16)])

        rdesc.wait()
        return carry

    lax.fori_loop(0, NPAIR, pair_b, 0)


def _sc_attention(comb, na_p, nb_p, tgt_p, sa, qa, sb, qb, wt, qt):
    mesh = plsc.VectorSubcoreMesh(core_axis_name="c", subcore_axis_name="s",
                                  num_cores=NC, num_subcores=NS)
    f32, i32 = jnp.float32, jnp.int32
    kern = functools.partial(
        pl.kernel,
        out_type=jax.ShapeDtypeStruct((BP, D), f32),
        mesh=mesh,
        compiler_params=pltpu.CompilerParams(needs_layout_passes=False,
                                             disable_bounds_checks=True),
        scratch_types=[
            pltpu.VMEM((N,), f32), pltpu.VMEM((N,), f32),
            pltpu.VMEM((N,), f32), pltpu.VMEM((N,), f32),
            pltpu.VMEM((N,), f32), pltpu.VMEM((N,), f32),
            pltpu.VMEM((NPW * K2,), i32), pltpu.VMEM((NPW * K2,), i32),
            pltpu.VMEM((NPW,), i32),
            pltpu.VMEM((NPW * 16,), f32),
            pltpu.VMEM((80, D), f32),
            pltpu.VMEM((96,), i32),
            pltpu.VMEM((16, D), f32),
            pltpu.SemaphoreType.DMA,
        ],
    )(_sc_body)
    return kern(comb, na_p, nb_p, tgt_p, sa, qa, sb, qb, wt, qt)


def kernel(target_ids, feats_A, feats_B, neigh_ids_A, neigh_ids_B,
           type_attn_query, node_attn_w, proj_w, proj_b):
    i32 = jnp.int32
    comb = jnp.concatenate([feats_A, feats_B], axis=0)

    q = type_attn_query[0]
    w = node_attn_w[0]
    wc = jnp.stack([w[D:], q[D:], w[:D], q[:D]], axis=1)
    wc128 = jnp.pad(wc, ((0, 0), (0, 124)))

    scal = _proj_scalars(comb, wc128)
    sa, qa = scal[:N, 0], scal[:N, 1]
    wt, qt = scal[:N, 2], scal[:N, 3]
    sb, qb = scal[N:, 0], scal[N:, 1]

    pad = BP - N
    tgt_p = jnp.pad(target_ids.astype(i32), (0, pad))
    na_p = jnp.pad(neigh_ids_A.astype(i32), ((0, pad), (0, 0))).reshape(-1)
    nb_p = jnp.pad(neigh_ids_B.astype(i32), ((0, pad), (0, 0))).reshape(-1)

    out_pre = _sc_attention(comb, na_p, nb_p, tgt_p,
                            sa, qa, sb, qb, wt, qt)

    y = _out_proj(out_pre, proj_w, proj_b.reshape(1, D))
    return y[:N]


# speedup vs baseline: 2.3133x; 1.0102x over previous
import functools

import jax
import jax.numpy as jnp
from jax import lax
from jax.experimental import pallas as pl
from jax.experimental.pallas import tpu as pltpu
from jax.experimental.pallas import tpu_sc as plsc

N = 10000
D = 512
K2 = 8
NC, NS = 2, 16
NW = NC * NS
BP = 10240
NPW = BP // NW
NG = NPW // 16


def _lrelu(x):
    return jnp.where(x >= 0, x, x * 0.2)


def _proj_scal_body(x_ref, w_ref, o_ref):
    o_ref[...] = jnp.dot(x_ref[...], w_ref[...],
                         preferred_element_type=jnp.float32)


def _proj_scalars(comb, wc128):
    grid = 10
    blk = (2 * N) // grid
    return pl.pallas_call(
        _proj_scal_body,
        grid=(grid,),
        in_specs=[
            pl.BlockSpec((blk, D), lambda i: (i, 0)),
            pl.BlockSpec((D, 128), lambda i: (0, 0)),
        ],
        out_specs=pl.BlockSpec((blk, 128), lambda i: (i, 0)),
        out_shape=jax.ShapeDtypeStruct((2 * N, 128), jnp.float32),
    )(comb, wc128)


def _out_proj_body(x_ref, w_ref, b_ref, o_ref):
    acc = lax.dot_general(x_ref[...], w_ref[...],
                          (((1,), (1,)), ((), ())),
                          preferred_element_type=jnp.float32)
    o_ref[...] = acc + b_ref[...]


def _out_proj(x, w, b):
    grid = BP // 512
    return pl.pallas_call(
        _out_proj_body,
        grid=(grid,),
        in_specs=[
            pl.BlockSpec((512, D), lambda i: (i, 0)),
            pl.BlockSpec((D, D), lambda i: (0, 0)),
            pl.BlockSpec((1, D), lambda i: (0, 0)),
        ],
        out_specs=pl.BlockSpec((512, D), lambda i: (i, 0)),
        out_shape=jax.ShapeDtypeStruct((BP, D), jnp.float32),
    )(x, w, b)


def _sc_body(comb_hbm, na_hbm, nb_hbm, tgt_hbm,
             sa_h, qa_h, sb_h, qb_h, wt_h, qt_h,
             out_hbm,
             tsa, tqa, tsb, tqb, twt, tqt,
             nav, nbv, tgtv, betv,
             rowsb, idxb, outb, rsem):
    wid = lax.axis_index("s") * NC + lax.axis_index("c")
    base = wid * NPW

    pltpu.sync_copy(sa_h, tsa)
    pltpu.sync_copy(qa_h, tqa)
    pltpu.sync_copy(sb_h, tsb)
    pltpu.sync_copy(qb_h, tqb)
    pltpu.sync_copy(wt_h, twt)
    pltpu.sync_copy(qt_h, tqt)
    pltpu.sync_copy(na_hbm.at[pl.ds(base * K2, NPW * K2)], nav)
    pltpu.sync_copy(nb_hbm.at[pl.ds(base * K2, NPW * K2)], nbv)
    pltpu.sync_copy(tgt_hbm.at[pl.ds(base, NPW)], tgtv)

    iota = lax.broadcasted_iota(jnp.int32, (16,), 0)

    def group_a(g, carry):
        gb = g * 16
        tgt = tgtv[pl.ds(gb, 16)]
        t_w = plsc.load_gather(twt, [tgt])
        t_q = plsc.load_gather(tqt, [tgt])
        qacc_a = jnp.zeros((16,), jnp.float32)
        qacc_b = jnp.zeros((16,), jnp.float32)
        ek = []
        for k in range(K2):
            ids = plsc.load_gather(nav, [iota * K2 + (gb * K2 + k)])
            qacc_a = qacc_a + plsc.load_gather(tqa, [ids])
            s = plsc.load_gather(tsa, [ids])
            ek.append(jnp.exp(_lrelu(t_w + s)))
        for k in range(K2):
            ids = plsc.load_gather(nbv, [iota * K2 + (gb * K2 + k)])
            qacc_b = qacc_b + plsc.load_gather(tqb, [ids])
            s = plsc.load_gather(tsb, [ids])
            ek.append(jnp.exp(_lrelu(t_w + s)))
        log_a = _lrelu(t_q + qacc_a * (1.0 / K2))
        log_b = _lrelu(t_q + qacc_b * (1.0 / K2))
        m = jnp.maximum(log_a, log_b)
        ea = jnp.exp(log_a - m)
        eb = jnp.exp(log_b - m)
        inv = 1.0 / (ea + eb)
        al_a = ea * inv
        al_b = eb * inv
        u = [ek[k] * al_a for k in range(K2)] + \
            [ek[K2 + k] * al_b for k in range(K2)]
        mu = u[0]
        for k in range(1, 16):
            mu = jnp.maximum(mu, u[k])
        w = [jnp.exp(u[k] - mu) for k in range(16)]
        ssum = w[0]
        for k in range(1, 16):
            ssum = ssum + w[k]
        inv_s = 1.0 / ssum
        for k in range(16):
            plsc.store_scatter(betv, [iota * 16 + (gb * 16 + k)],
                               w[k] * inv_s)
        return carry

    lax.fori_loop(0, NG, group_a, 0)

    def _stage_pair_idx(p, slot):
        n0 = 2 * p
        ia = plsc.load_gather(nav, [n0 * K2 + (iota & (K2 - 1))])
        ib = plsc.load_gather(nbv, [n0 * K2 + (iota & (K2 - 1))]) + N
        idxb[pl.ds(slot * 48, 16)] = jnp.where(iota < K2, ia, ib)
        ia = plsc.load_gather(nav, [(n0 + 1) * K2 + (iota & (K2 - 1))])
        ib = plsc.load_gather(nbv, [(n0 + 1) * K2 + (iota & (K2 - 1))]) + N
        idxb[pl.ds(slot * 48 + 16, 16)] = jnp.where(iota < K2, ia, ib)
        tv = plsc.load_gather(tgtv, [jnp.minimum(n0 + iota, NPW - 1)])
        idxb[pl.ds(slot * 48 + 32, 16)] = tv

    def _fire_pair(slot):
        return pltpu.async_copy(
            comb_hbm.at[idxb.at[pl.ds(slot * 48, 40)]],
            rowsb.at[pl.ds(slot * 40, 40)], rsem)

    def _compute_node(n, rowbase, tgtrow, orow):
        for h in range(4):
            c0 = h * (D // 64)
            accs = [rowsb[tgtrow, pl.ds((c0 + c) * 16, 16)]
                    for c in range(D // 64)]
            for k in range(16):
                bk = plsc.load_gather(
                    betv, [jnp.full((16,), n * 16 + k, jnp.int32)])
                for c in range(D // 64):
                    accs[c] = accs[c] + bk * rowsb[rowbase + k,
                                                   pl.ds((c0 + c) * 16, 16)]
            for c in range(D // 64):
                outb[orow, pl.ds((c0 + c) * 16, 16)] = accs[c]

    def _compute_pair(p, slot):
        rb = slot * 40
        _compute_node(2 * p, rb, rb + 32, (p & 7) * 2)
        _compute_node(2 * p + 1, rb + 16, rb + 33, (p & 7) * 2 + 1)

    NPAIR = NPW // 2
    _stage_pair_idx(0, 0)
    _fire_pair(0).wait()

    def pair2_b(q, carry):
        p0 = 2 * q
        p1 = 2 * q + 1
        _stage_pair_idx(p1, 1)
        d1 = _fire_pair(1)
        _compute_pair(p0, 0)
        d1.wait()

        pn = jnp.minimum(p1 + 1, NPAIR - 1)
        _stage_pair_idx(pn, 0)
        d0 = _fire_pair(0)
        _compute_pair(p1, 1)

        @pl.when((q & 3) == 3)
        def _():
            pltpu.sync_copy(outb,
                            out_hbm.at[pl.ds(base + (q // 4) * 16, 16)])

        d0.wait()
        return carry

    lax.fori_loop(0, NPAIR // 2, pair2_b, 0)


def _sc_attention(comb, na_p, nb_p, tgt_p, sa, qa, sb, qb, wt, qt):
    mesh = plsc.VectorSubcoreMesh(core_axis_name="c", subcore_axis_name="s",
                                  num_cores=NC, num_subcores=NS)
    f32, i32 = jnp.float32, jnp.int32
    kern = functools.partial(
        pl.kernel,
        out_type=jax.ShapeDtypeStruct((BP, D), f32),
        mesh=mesh,
        compiler_params=pltpu.CompilerParams(needs_layout_passes=False,
                                             disable_bounds_checks=True),
        scratch_types=[
            pltpu.VMEM((N,), f32), pltpu.VMEM((N,), f32),
            pltpu.VMEM((N,), f32), pltpu.VMEM((N,), f32),
            pltpu.VMEM((N,), f32), pltpu.VMEM((N,), f32),
            pltpu.VMEM((NPW * K2,), i32), pltpu.VMEM((NPW * K2,), i32),
            pltpu.VMEM((NPW,), i32),
            pltpu.VMEM((NPW * 16,), f32),
            pltpu.VMEM((80, D), f32),
            pltpu.VMEM((96,), i32),
            pltpu.VMEM((16, D), f32),
            pltpu.SemaphoreType.DMA,
        ],
    )(_sc_body)
    return kern(comb, na_p, nb_p, tgt_p, sa, qa, sb, qb, wt, qt)


def kernel(target_ids, feats_A, feats_B, neigh_ids_A, neigh_ids_B,
           type_attn_query, node_attn_w, proj_w, proj_b):
    i32 = jnp.int32
    comb = jnp.concatenate([feats_A, feats_B], axis=0)

    q = type_attn_query[0]
    w = node_attn_w[0]
    wc = jnp.stack([w[D:], q[D:], w[:D], q[:D]], axis=1)
    wc128 = jnp.pad(wc, ((0, 0), (0, 124)))

    scal = _proj_scalars(comb, wc128)
    sa, qa = scal[:N, 0], scal[:N, 1]
    wt, qt = scal[:N, 2], scal[:N, 3]
    sb, qb = scal[N:, 0], scal[N:, 1]

    pad = BP - N
    tgt_p = jnp.pad(target_ids.astype(i32), (0, pad))
    na_p = jnp.pad(neigh_ids_A.astype(i32), ((0, pad), (0, 0))).reshape(-1)
    nb_p = jnp.pad(neigh_ids_B.astype(i32), ((0, pad), (0, 0))).reshape(-1)

    out_pre = _sc_attention(comb, na_p, nb_p, tgt_p,
                            sa, qa, sb, qb, wt, qt)

    y = _out_proj(out_pre, proj_w, proj_b.reshape(1, D))
    return y[:N]


# speedup vs baseline: 2.3409x; 1.0120x over previous
import functools

import jax
import jax.numpy as jnp
from jax import lax
from jax.experimental import pallas as pl
from jax.experimental.pallas import tpu as pltpu
from jax.experimental.pallas import tpu_sc as plsc

N = 10000
D = 512
K2 = 8
NC, NS = 2, 16
NW = NC * NS
BP = 10240
NPW = BP // NW
NG = NPW // 16


def _lrelu(x):
    return jnp.where(x >= 0, x, x * 0.2)


def _proj_scal_body(x_ref, w_ref, o_ref):
    o_ref[...] = jnp.dot(x_ref[...], w_ref[...],
                         preferred_element_type=jnp.float32)


def _proj_scalars(comb, wc128):
    grid = 10
    blk = (2 * N) // grid
    return pl.pallas_call(
        _proj_scal_body,
        grid=(grid,),
        in_specs=[
            pl.BlockSpec((blk, D), lambda i: (i, 0)),
            pl.BlockSpec((D, 128), lambda i: (0, 0)),
        ],
        out_specs=pl.BlockSpec((blk, 128), lambda i: (i, 0)),
        out_shape=jax.ShapeDtypeStruct((2 * N, 128), jnp.float32),
    )(comb, wc128)


def _out_proj_body(x_ref, w_ref, b_ref, o_ref):
    acc = lax.dot_general(x_ref[...], w_ref[...],
                          (((1,), (1,)), ((), ())),
                          preferred_element_type=jnp.float32)
    o_ref[...] = acc + b_ref[...]


def _out_proj(x, w, b):
    grid = BP // 512
    return pl.pallas_call(
        _out_proj_body,
        grid=(grid,),
        in_specs=[
            pl.BlockSpec((512, D), lambda i: (i, 0)),
            pl.BlockSpec((D, D), lambda i: (0, 0)),
            pl.BlockSpec((1, D), lambda i: (0, 0)),
        ],
        out_specs=pl.BlockSpec((512, D), lambda i: (i, 0)),
        out_shape=jax.ShapeDtypeStruct((BP, D), jnp.float32),
    )(x, w, b)


def _sc_body(comb_hbm, na_hbm, nb_hbm, tgt_hbm,
             sa_h, qa_h, sb_h, qb_h, wt_h, qt_h,
             out_hbm,
             tsa, tqa, tsb, tqb, twt, tqt,
             nav, nbv, tgtv, betv,
             rowsb, idxb, outb, rsem):
    wid = lax.axis_index("s") * NC + lax.axis_index("c")
    base = wid * NPW

    pltpu.sync_copy(sa_h, tsa)
    pltpu.sync_copy(qa_h, tqa)
    pltpu.sync_copy(sb_h, tsb)
    pltpu.sync_copy(qb_h, tqb)
    pltpu.sync_copy(wt_h, twt)
    pltpu.sync_copy(qt_h, tqt)
    pltpu.sync_copy(na_hbm.at[pl.ds(base * K2, NPW * K2)], nav)
    pltpu.sync_copy(nb_hbm.at[pl.ds(base * K2, NPW * K2)], nbv)
    pltpu.sync_copy(tgt_hbm.at[pl.ds(base, NPW)], tgtv)

    iota = lax.broadcasted_iota(jnp.int32, (16,), 0)

    def group_a(g, carry):
        gb = g * 16
        tgt = tgtv[pl.ds(gb, 16)]
        t_w = plsc.load_gather(twt, [tgt])
        t_q = plsc.load_gather(tqt, [tgt])
        qacc_a = jnp.zeros((16,), jnp.float32)
        qacc_b = jnp.zeros((16,), jnp.float32)
        ek = []
        for k in range(K2):
            ids = plsc.load_gather(nav, [iota * K2 + (gb * K2 + k)])
            qacc_a = qacc_a + plsc.load_gather(tqa, [ids])
            s = plsc.load_gather(tsa, [ids])
            ek.append(jnp.exp(_lrelu(t_w + s)))
        for k in range(K2):
            ids = plsc.load_gather(nbv, [iota * K2 + (gb * K2 + k)])
            qacc_b = qacc_b + plsc.load_gather(tqb, [ids])
            s = plsc.load_gather(tsb, [ids])
            ek.append(jnp.exp(_lrelu(t_w + s)))
        log_a = _lrelu(t_q + qacc_a * (1.0 / K2))
        log_b = _lrelu(t_q + qacc_b * (1.0 / K2))
        m = jnp.maximum(log_a, log_b)
        ea = jnp.exp(log_a - m)
        eb = jnp.exp(log_b - m)
        inv = 1.0 / (ea + eb)
        al_a = ea * inv
        al_b = eb * inv
        u = [ek[k] * al_a for k in range(K2)] + \
            [ek[K2 + k] * al_b for k in range(K2)]
        mu = u[0]
        for k in range(1, 16):
            mu = jnp.maximum(mu, u[k])
        w = [jnp.exp(u[k] - mu) for k in range(16)]
        ssum = w[0]
        for k in range(1, 16):
            ssum = ssum + w[k]
        inv_s = 1.0 / ssum
        for k in range(16):
            plsc.store_scatter(betv, [iota * 16 + (gb * 16 + k)],
                               w[k] * inv_s)
        return carry

    lax.fori_loop(0, NG, group_a, 0)

    def _stage_pair_idx(p, slot):
        n0 = 2 * p
        ia = plsc.load_gather(nav, [n0 * K2 + (iota & (K2 - 1))])
        ib = plsc.load_gather(nbv, [n0 * K2 + (iota & (K2 - 1))]) + N
        idxb[pl.ds(slot * 48, 16)] = jnp.where(iota < K2, ia, ib)
        ia = plsc.load_gather(nav, [(n0 + 1) * K2 + (iota & (K2 - 1))])
        ib = plsc.load_gather(nbv, [(n0 + 1) * K2 + (iota & (K2 - 1))]) + N
        idxb[pl.ds(slot * 48 + 16, 16)] = jnp.where(iota < K2, ia, ib)
        tv = plsc.load_gather(tgtv, [jnp.minimum(n0 + iota, NPW - 1)])
        idxb[pl.ds(slot * 48 + 32, 16)] = tv

    def _fire_pair(slot):
        return pltpu.async_copy(
            comb_hbm.at[idxb.at[pl.ds(slot * 48, 40)]],
            rowsb.at[pl.ds(slot * 40, 40)], rsem)

    def _compute_node(n, rowbase, tgtrow, orow):
        bvec = betv[pl.ds(n * 16, 16)]
        dn = lax.GatherDimensionNumbers(offset_dims=(),
                                        collapsed_slice_dims=(0,),
                                        start_index_map=(0,))
        bks = [lax.gather(bvec, jnp.full((16, 1), k, jnp.int32), dn,
                          slice_sizes=(1,),
                          mode=lax.GatherScatterMode.PROMISE_IN_BOUNDS)
               for k in range(16)]
        for h in range(4):
            c0 = h * (D // 64)
            accs = [rowsb[tgtrow, pl.ds((c0 + c) * 16, 16)]
                    for c in range(D // 64)]
            for k in range(16):
                for c in range(D // 64):
                    accs[c] = accs[c] + bks[k] * rowsb[rowbase + k,
                                                       pl.ds((c0 + c) * 16,
                                                             16)]
            for c in range(D // 64):
                outb[orow, pl.ds((c0 + c) * 16, 16)] = accs[c]

    def _compute_pair(p, slot):
        rb = slot * 40
        _compute_node(2 * p, rb, rb + 32, (p & 7) * 2)
        _compute_node(2 * p + 1, rb + 16, rb + 33, (p & 7) * 2 + 1)

    NPAIR = NPW // 2
    _stage_pair_idx(0, 0)
    _fire_pair(0).wait()

    def pair2_b(q, carry):
        p0 = 2 * q
        p1 = 2 * q + 1
        _stage_pair_idx(p1, 1)
        d1 = _fire_pair(1)
        _compute_pair(p0, 0)
        d1.wait()

        pn = jnp.minimum(p1 + 1, NPAIR - 1)
        _stage_pair_idx(pn, 0)
        d0 = _fire_pair(0)
        _compute_pair(p1, 1)

        @pl.when((q & 3) == 3)
        def _():
            pltpu.sync_copy(outb,
                            out_hbm.at[pl.ds(base + (q // 4) * 16, 16)])

        d0.wait()
        return carry

    lax.fori_loop(0, NPAIR // 2, pair2_b, 0)


def _sc_attention(comb, na_p, nb_p, tgt_p, sa, qa, sb, qb, wt, qt):
    mesh = plsc.VectorSubcoreMesh(core_axis_name="c", subcore_axis_name="s",
                                  num_cores=NC, num_subcores=NS)
    f32, i32 = jnp.float32, jnp.int32
    kern = functools.partial(
        pl.kernel,
        out_type=jax.ShapeDtypeStruct((BP, D), f32),
        mesh=mesh,
        compiler_params=pltpu.CompilerParams(needs_layout_passes=False,
                                             disable_bounds_checks=True),
        scratch_types=[
            pltpu.VMEM((N,), f32), pltpu.VMEM((N,), f32),
            pltpu.VMEM((N,), f32), pltpu.VMEM((N,), f32),
            pltpu.VMEM((N,), f32), pltpu.VMEM((N,), f32),
            pltpu.VMEM((NPW * K2,), i32), pltpu.VMEM((NPW * K2,), i32),
            pltpu.VMEM((NPW,), i32),
            pltpu.VMEM((NPW * 16,), f32),
            pltpu.VMEM((80, D), f32),
            pltpu.VMEM((96,), i32),
            pltpu.VMEM((16, D), f32),
            pltpu.SemaphoreType.DMA,
        ],
    )(_sc_body)
    return kern(comb, na_p, nb_p, tgt_p, sa, qa, sb, qb, wt, qt)


def kernel(target_ids, feats_A, feats_B, neigh_ids_A, neigh_ids_B,
           type_attn_query, node_attn_w, proj_w, proj_b):
    i32 = jnp.int32
    comb = jnp.concatenate([feats_A, feats_B], axis=0)

    q = type_attn_query[0]
    w = node_attn_w[0]
    wc = jnp.stack([w[D:], q[D:], w[:D], q[:D]], axis=1)
    wc128 = jnp.pad(wc, ((0, 0), (0, 124)))

    scal = _proj_scalars(comb, wc128)
    sa, qa = scal[:N, 0], scal[:N, 1]
    wt, qt = scal[:N, 2], scal[:N, 3]
    sb, qb = scal[N:, 0], scal[N:, 1]

    pad = BP - N
    tgt_p = jnp.pad(target_ids.astype(i32), (0, pad))
    na_p = jnp.pad(neigh_ids_A.astype(i32), ((0, pad), (0, 0))).reshape(-1)
    nb_p = jnp.pad(neigh_ids_B.astype(i32), ((0, pad), (0, 0))).reshape(-1)

    out_pre = _sc_attention(comb, na_p, nb_p, tgt_p,
                            sa, qa, sb, qb, wt, qt)

    y = _out_proj(out_pre, proj_w, proj_b.reshape(1, D))
    return y[:N]


# speedup vs baseline: 2.3803x; 1.0168x over previous
import functools

import jax
import jax.numpy as jnp
from jax import lax
from jax.experimental import pallas as pl
from jax.experimental.pallas import tpu as pltpu
from jax.experimental.pallas import tpu_sc as plsc

N = 10000
D = 512
K2 = 8
NC, NS = 2, 16
NW = NC * NS
BP = 10240
NPW = BP // NW
NG = NPW // 16


def _lrelu(x):
    return jnp.where(x >= 0, x, x * 0.2)


def _proj_scal_body(x_ref, w_ref, o_ref):
    o_ref[...] = jnp.dot(x_ref[...], w_ref[...],
                         preferred_element_type=jnp.float32)


def _proj_scalars(comb, wc128):
    grid = 10
    blk = (2 * N) // grid
    return pl.pallas_call(
        _proj_scal_body,
        grid=(grid,),
        in_specs=[
            pl.BlockSpec((blk, D), lambda i: (i, 0)),
            pl.BlockSpec((D, 128), lambda i: (0, 0)),
        ],
        out_specs=pl.BlockSpec((blk, 128), lambda i: (i, 0)),
        out_shape=jax.ShapeDtypeStruct((2 * N, 128), jnp.float32),
    )(comb, wc128)


def _out_proj_body(x_ref, w_ref, b_ref, o_ref):
    acc = lax.dot_general(x_ref[...], w_ref[...],
                          (((1,), (1,)), ((), ())),
                          preferred_element_type=jnp.float32)
    o_ref[...] = acc + b_ref[...]


def _out_proj(x, w, b):
    grid = BP // 512
    return pl.pallas_call(
        _out_proj_body,
        grid=(grid,),
        in_specs=[
            pl.BlockSpec((512, D), lambda i: (i, 0)),
            pl.BlockSpec((D, D), lambda i: (0, 0)),
            pl.BlockSpec((1, D), lambda i: (0, 0)),
        ],
        out_specs=pl.BlockSpec((512, D), lambda i: (i, 0)),
        out_shape=jax.ShapeDtypeStruct((BP, D), jnp.float32),
    )(x, w, b)


def _sc_body(comb_hbm, na_hbm, nb_hbm, tgt_hbm,
             sa_h, qa_h, sb_h, qb_h, wt_h, qt_h,
             out_hbm,
             tsa, tqa, tsb, tqb, twt, tqt,
             nav, nbv, tgtv, betv,
             rowsb, idxb, outb, rsem):
    wid = lax.axis_index("s") * NC + lax.axis_index("c")
    base = wid * NPW

    pltpu.sync_copy(sa_h, tsa)
    pltpu.sync_copy(qa_h, tqa)
    pltpu.sync_copy(sb_h, tsb)
    pltpu.sync_copy(qb_h, tqb)
    pltpu.sync_copy(wt_h, twt)
    pltpu.sync_copy(qt_h, tqt)
    pltpu.sync_copy(na_hbm.at[pl.ds(base * K2, NPW * K2)], nav)
    pltpu.sync_copy(nb_hbm.at[pl.ds(base * K2, NPW * K2)], nbv)
    pltpu.sync_copy(tgt_hbm.at[pl.ds(base, NPW)], tgtv)

    iota = lax.broadcasted_iota(jnp.int32, (16,), 0)

    def group_a(g, carry):
        gb = g * 16
        tgt = tgtv[pl.ds(gb, 16)]
        t_w = plsc.load_gather(twt, [tgt])
        t_q = plsc.load_gather(tqt, [tgt])
        qacc_a = jnp.zeros((16,), jnp.float32)
        qacc_b = jnp.zeros((16,), jnp.float32)
        ek = []
        for k in range(K2):
            ids = plsc.load_gather(nav, [iota * K2 + (gb * K2 + k)])
            qacc_a = qacc_a + plsc.load_gather(tqa, [ids])
            s = plsc.load_gather(tsa, [ids])
            ek.append(jnp.exp(_lrelu(t_w + s)))
        for k in range(K2):
            ids = plsc.load_gather(nbv, [iota * K2 + (gb * K2 + k)])
            qacc_b = qacc_b + plsc.load_gather(tqb, [ids])
            s = plsc.load_gather(tsb, [ids])
            ek.append(jnp.exp(_lrelu(t_w + s)))
        log_a = _lrelu(t_q + qacc_a * (1.0 / K2))
        log_b = _lrelu(t_q + qacc_b * (1.0 / K2))
        m = jnp.maximum(log_a, log_b)
        ea = jnp.exp(log_a - m)
        eb = jnp.exp(log_b - m)
        inv = 1.0 / (ea + eb)
        al_a = ea * inv
        al_b = eb * inv
        u = [ek[k] * al_a for k in range(K2)] + \
            [ek[K2 + k] * al_b for k in range(K2)]
        mu = u[0]
        for k in range(1, 16):
            mu = jnp.maximum(mu, u[k])
        w = [jnp.exp(u[k] - mu) for k in range(16)]
        ssum = w[0]
        for k in range(1, 16):
            ssum = ssum + w[k]
        inv_s = 1.0 / ssum
        for k in range(16):
            plsc.store_scatter(betv, [iota * 16 + (gb * 16 + k)],
                               w[k] * inv_s)
        return carry

    lax.fori_loop(0, NG, group_a, 0)

    def _stage_pair_idx(p, slot):
        n0 = 2 * p
        ia = plsc.load_gather(nav, [n0 * K2 + (iota & (K2 - 1))])
        ib = plsc.load_gather(nbv, [n0 * K2 + (iota & (K2 - 1))]) + N
        idxb[pl.ds(slot * 48, 16)] = jnp.where(iota < K2, ia, ib)
        ia = plsc.load_gather(nav, [(n0 + 1) * K2 + (iota & (K2 - 1))])
        ib = plsc.load_gather(nbv, [(n0 + 1) * K2 + (iota & (K2 - 1))]) + N
        idxb[pl.ds(slot * 48 + 16, 16)] = jnp.where(iota < K2, ia, ib)
        tv = plsc.load_gather(tgtv, [jnp.minimum(n0 + iota, NPW - 1)])
        idxb[pl.ds(slot * 48 + 32, 16)] = tv

    def _fire_pair(slot):
        return pltpu.async_copy(
            comb_hbm.at[idxb.at[pl.ds(slot * 48, 40)]],
            rowsb.at[pl.ds(slot * 40, 40)], rsem)

    _dn = lax.GatherDimensionNumbers(offset_dims=(),
                                     collapsed_slice_dims=(0,),
                                     start_index_map=(0,))

    def _compute_node(n, rowbase, tgtrow, orow):
        bvec = betv[pl.ds(n * 16, 16)]
        init = tuple(rowsb[tgtrow, pl.ds(c * 16, 16)]
                     for c in range(D // 16))

        def kbody(k, accs):
            bk = lax.gather(bvec, jnp.full((16, 1), k, jnp.int32), _dn,
                            slice_sizes=(1,),
                            mode=lax.GatherScatterMode.PROMISE_IN_BOUNDS)
            row = rowbase + k
            return tuple(accs[c] + bk * rowsb[row, pl.ds(c * 16, 16)]
                         for c in range(D // 16))

        accs = lax.fori_loop(0, 16, kbody, init)
        for c in range(D // 16):
            outb[orow, pl.ds(c * 16, 16)] = accs[c]

    def _compute_pair(p, slot):
        rb = slot * 40
        _compute_node(2 * p, rb, rb + 32, (p & 7) * 2)
        _compute_node(2 * p + 1, rb + 16, rb + 33, (p & 7) * 2 + 1)

    NPAIR = NPW // 2
    _stage_pair_idx(0, 0)
    _fire_pair(0).wait()

    def pair2_b(q, carry):
        p0 = 2 * q
        p1 = 2 * q + 1
        _stage_pair_idx(p1, 1)
        d1 = _fire_pair(1)
        _compute_pair(p0, 0)
        d1.wait()

        pn = jnp.minimum(p1 + 1, NPAIR - 1)
        _stage_pair_idx(pn, 0)
        d0 = _fire_pair(0)
        _compute_pair(p1, 1)

        @pl.when((q & 3) == 3)
        def _():
            pltpu.sync_copy(outb,
                            out_hbm.at[pl.ds(base + (q // 4) * 16, 16)])

        d0.wait()
        return carry

    lax.fori_loop(0, NPAIR // 2, pair2_b, 0)


def _sc_attention(comb, na_p, nb_p, tgt_p, sa, qa, sb, qb, wt, qt):
    mesh = plsc.VectorSubcoreMesh(core_axis_name="c", subcore_axis_name="s",
                                  num_cores=NC, num_subcores=NS)
    f32, i32 = jnp.float32, jnp.int32
    kern = functools.partial(
        pl.kernel,
        out_type=jax.ShapeDtypeStruct((BP, D), f32),
        mesh=mesh,
        compiler_params=pltpu.CompilerParams(needs_layout_passes=False,
                                             disable_bounds_checks=True),
        scratch_types=[
            pltpu.VMEM((N,), f32), pltpu.VMEM((N,), f32),
            pltpu.VMEM((N,), f32), pltpu.VMEM((N,), f32),
            pltpu.VMEM((N,), f32), pltpu.VMEM((N,), f32),
            pltpu.VMEM((NPW * K2,), i32), pltpu.VMEM((NPW * K2,), i32),
            pltpu.VMEM((NPW,), i32),
            pltpu.VMEM((NPW * 16,), f32),
            pltpu.VMEM((80, D), f32),
            pltpu.VMEM((96,), i32),
            pltpu.VMEM((16, D), f32),
            pltpu.SemaphoreType.DMA,
        ],
    )(_sc_body)
    return kern(comb, na_p, nb_p, tgt_p, sa, qa, sb, qb, wt, qt)


def kernel(target_ids, feats_A, feats_B, neigh_ids_A, neigh_ids_B,
           type_attn_query, node_attn_w, proj_w, proj_b):
    i32 = jnp.int32
    comb = jnp.concatenate([feats_A, feats_B], axis=0)

    q = type_attn_query[0]
    w = node_attn_w[0]
    wc = jnp.stack([w[D:], q[D:], w[:D], q[:D]], axis=1)
    wc128 = jnp.pad(wc, ((0, 0), (0, 124)))

    scal = _proj_scalars(comb, wc128)
    sa, qa = scal[:N, 0], scal[:N, 1]
    wt, qt = scal[:N, 2], scal[:N, 3]
    sb, qb = scal[N:, 0], scal[N:, 1]

    pad = BP - N
    tgt_p = jnp.pad(target_ids.astype(i32), (0, pad))
    na_p = jnp.pad(neigh_ids_A.astype(i32), ((0, pad), (0, 0))).reshape(-1)
    nb_p = jnp.pad(neigh_ids_B.astype(i32), ((0, pad), (0, 0))).reshape(-1)

    out_pre = _sc_attention(comb, na_p, nb_p, tgt_p,
                            sa, qa, sb, qb, wt, qt)

    y = _out_proj(out_pre, proj_w, proj_b.reshape(1, D))
    return y[:N]


# speedup vs baseline: 2.5160x; 1.0570x over previous
import functools

import jax
import jax.numpy as jnp
from jax import lax
from jax.experimental import pallas as pl
from jax.experimental.pallas import tpu as pltpu
from jax.experimental.pallas import tpu_sc as plsc

N = 10000
D = 512
K2 = 8
NC, NS = 2, 16
NW = NC * NS
BP = 10240
NPW = BP // NW
NG = NPW // 16


def _lrelu(x):
    return jnp.where(x >= 0, x, x * 0.2)


def _proj_scal_body(x_ref, w_ref, o_ref):
    o_ref[...] = jnp.dot(x_ref[...], w_ref[...],
                         preferred_element_type=jnp.float32)


def _proj_scalars(comb, wc128):
    grid = 10
    blk = (2 * N) // grid
    return pl.pallas_call(
        _proj_scal_body,
        grid=(grid,),
        in_specs=[
            pl.BlockSpec((blk, D), lambda i: (i, 0)),
            pl.BlockSpec((D, 128), lambda i: (0, 0)),
        ],
        out_specs=pl.BlockSpec((blk, 128), lambda i: (i, 0)),
        out_shape=jax.ShapeDtypeStruct((2 * N, 128), jnp.float32),
    )(comb, wc128)


def _out_proj_body(x_ref, w_ref, b_ref, o_ref):
    acc = lax.dot_general(x_ref[...], w_ref[...],
                          (((1,), (1,)), ((), ())),
                          preferred_element_type=jnp.float32)
    o_ref[...] = acc + b_ref[...]


def _out_proj(x, w, b):
    grid = BP // 512
    return pl.pallas_call(
        _out_proj_body,
        grid=(grid,),
        in_specs=[
            pl.BlockSpec((512, D), lambda i: (i, 0)),
            pl.BlockSpec((D, D), lambda i: (0, 0)),
            pl.BlockSpec((1, D), lambda i: (0, 0)),
        ],
        out_specs=pl.BlockSpec((512, D), lambda i: (i, 0)),
        out_shape=jax.ShapeDtypeStruct((BP, D), jnp.float32),
    )(x, w, b)


def _sc_body(comb_hbm, na_hbm, nb_hbm, tgt_hbm,
             sa_h, qa_h, sb_h, qb_h, wt_h, qt_h,
             out_hbm,
             tsa, tqa, tsb, tqb, twt, tqt,
             nav, nbv, tgtv, betv,
             rowsb, tgtb, outb, rsem, osem, tsem):
    wid = lax.axis_index("s") * NC + lax.axis_index("c")
    base = wid * NPW

    pltpu.sync_copy(sa_h, tsa)
    pltpu.sync_copy(qa_h, tqa)
    pltpu.sync_copy(sb_h, tsb)
    pltpu.sync_copy(qb_h, tqb)
    pltpu.sync_copy(wt_h, twt)
    pltpu.sync_copy(qt_h, tqt)
    pltpu.sync_copy(na_hbm.at[pl.ds(base * K2, NPW * K2)], nav)
    pltpu.sync_copy(nb_hbm.at[pl.ds(base * K2, NPW * K2)], nbv)
    pltpu.sync_copy(tgt_hbm.at[pl.ds(base, NPW)], tgtv)

    iota = lax.broadcasted_iota(jnp.int32, (16,), 0)

    def group_a(g, carry):
        gb = g * 16
        tgt = tgtv[pl.ds(gb, 16)]
        t_w = plsc.load_gather(twt, [tgt])
        t_q = plsc.load_gather(tqt, [tgt])
        qacc_a = jnp.zeros((16,), jnp.float32)
        qacc_b = jnp.zeros((16,), jnp.float32)
        ek = []
        for k in range(K2):
            ids = plsc.load_gather(nav, [iota * K2 + (gb * K2 + k)])
            qacc_a = qacc_a + plsc.load_gather(tqa, [ids])
            s = plsc.load_gather(tsa, [ids])
            ek.append(jnp.exp(_lrelu(t_w + s)))
        for k in range(K2):
            ids = plsc.load_gather(nbv, [iota * K2 + (gb * K2 + k)])
            qacc_b = qacc_b + plsc.load_gather(tqb, [ids])
            s = plsc.load_gather(tsb, [ids])
            ek.append(jnp.exp(_lrelu(t_w + s)))
        log_a = _lrelu(t_q + qacc_a * (1.0 / K2))
        log_b = _lrelu(t_q + qacc_b * (1.0 / K2))
        m = jnp.maximum(log_a, log_b)
        ea = jnp.exp(log_a - m)
        eb = jnp.exp(log_b - m)
        inv = 1.0 / (ea + eb)
        al_a = ea * inv
        al_b = eb * inv
        u = [ek[k] * al_a for k in range(K2)] + \
            [ek[K2 + k] * al_b for k in range(K2)]
        mu = u[0]
        for k in range(1, 16):
            mu = jnp.maximum(mu, u[k])
        w = [jnp.exp(u[k] - mu) for k in range(16)]
        ssum = w[0]
        for k in range(1, 16):
            ssum = ssum + w[k]
        inv_s = 1.0 / ssum
        for k in range(16):
            plsc.store_scatter(betv, [iota * 16 + (gb * 16 + k)],
                               w[k] * inv_s)
        return carry

    lax.fori_loop(0, NG, group_a, 0)

    def _node_idx(n):
        ia = plsc.load_gather(nav, [n * K2 + (iota & (K2 - 1))])
        ib = plsc.load_gather(nbv, [n * K2 + (iota & (K2 - 1))]) + N
        return jnp.where(iota < K2, ia, ib)

    def _fire_node(n, sem):
        slot = n & 3
        return pltpu.async_copy(comb_hbm.at[_node_idx(n)],
                                rowsb.at[pl.ds(slot * 16, 16)], sem)

    def _fire_tgt(g, sem):
        tvec = tgtv[pl.ds(g * 16, 16)]
        return pltpu.async_copy(comb_hbm.at[tvec],
                                tgtb.at[pl.ds((g & 1) * 16, 16)], sem)

    _dn = lax.GatherDimensionNumbers(offset_dims=(),
                                     collapsed_slice_dims=(0,),
                                     start_index_map=(0,))

    def _compute_node(n, rowbase, tgtrow, orow):
        bvec = betv[pl.ds(n * 16, 16)]
        init = tuple(tgtb[tgtrow, pl.ds(c * 16, 16)]
                     for c in range(D // 16))

        def kbody(k, accs):
            bk = lax.gather(bvec, jnp.full((16, 1), k, jnp.int32), _dn,
                            slice_sizes=(1,),
                            mode=lax.GatherScatterMode.PROMISE_IN_BOUNDS)
            row = rowbase + k
            return tuple(accs[c] + bk * rowsb[row, pl.ds(c * 16, 16)]
                         for c in range(D // 16))

        accs = lax.fori_loop(0, 16, kbody, init)
        for c in range(D // 16):
            outb[orow, pl.ds(c * 16, 16)] = accs[c]

    _fire_tgt(0, tsem).wait()
    _fire_node(0, rsem).wait()
    _fire_node(1, tsem).wait()

    def node2_b(q, carry):
        n0 = 2 * q
        d_e = _fire_node(jnp.minimum(n0 + 2, NPW - 1), rsem)
        d_o = _fire_node(jnp.minimum(n0 + 3, NPW - 1), osem)
        g = q // 8

        @pl.when((q & 7) == 0)
        def _():
            _fire_tgt(jnp.minimum(g + 1, NG - 1), tsem).wait()

        i0 = n0 - g * 16
        tb = (g & 1) * 16
        _compute_node(n0, (n0 & 3) * 16, tb + i0, n0 & 7)
        _compute_node(n0 + 1, ((n0 + 1) & 3) * 16, tb + i0 + 1,
                      (n0 + 1) & 7)

        @pl.when((q & 3) == 3)
        def _():
            pltpu.sync_copy(outb,
                            out_hbm.at[pl.ds(base + (q // 4) * 8, 8)])

        d_e.wait()
        d_o.wait()
        return carry

    lax.fori_loop(0, NPW // 2, node2_b, 0)


def _sc_attention(comb, na_p, nb_p, tgt_p, sa, qa, sb, qb, wt, qt):
    mesh = plsc.VectorSubcoreMesh(core_axis_name="c", subcore_axis_name="s",
                                  num_cores=NC, num_subcores=NS)
    f32, i32 = jnp.float32, jnp.int32
    kern = functools.partial(
        pl.kernel,
        out_type=jax.ShapeDtypeStruct((BP, D), f32),
        mesh=mesh,
        compiler_params=pltpu.CompilerParams(needs_layout_passes=False,
                                             disable_bounds_checks=True),
        scratch_types=[
            pltpu.VMEM((N,), f32), pltpu.VMEM((N,), f32),
            pltpu.VMEM((N,), f32), pltpu.VMEM((N,), f32),
            pltpu.VMEM((N,), f32), pltpu.VMEM((N,), f32),
            pltpu.VMEM((NPW * K2,), i32), pltpu.VMEM((NPW * K2,), i32),
            pltpu.VMEM((NPW,), i32),
            pltpu.VMEM((NPW * 16,), f32),
            pltpu.VMEM((64, D), f32),
            pltpu.VMEM((32, D), f32),
            pltpu.VMEM((8, D), f32),
            pltpu.SemaphoreType.DMA,
            pltpu.SemaphoreType.DMA,
            pltpu.SemaphoreType.DMA,
        ],
    )(_sc_body)
    return kern(comb, na_p, nb_p, tgt_p, sa, qa, sb, qb, wt, qt)


def kernel(target_ids, feats_A, feats_B, neigh_ids_A, neigh_ids_B,
           type_attn_query, node_attn_w, proj_w, proj_b):
    i32 = jnp.int32
    comb = jnp.concatenate([feats_A, feats_B], axis=0)

    q = type_attn_query[0]
    w = node_attn_w[0]
    wc = jnp.stack([w[D:], q[D:], w[:D], q[:D]], axis=1)
    wc128 = jnp.pad(wc, ((0, 0), (0, 124)))

    scal = _proj_scalars(comb, wc128)
    sa, qa = scal[:N, 0], scal[:N, 1]
    wt, qt = scal[:N, 2], scal[:N, 3]
    sb, qb = scal[N:, 0], scal[N:, 1]

    pad = BP - N
    tgt_p = jnp.pad(target_ids.astype(i32), (0, pad))
    na_p = jnp.pad(neigh_ids_A.astype(i32), ((0, pad), (0, 0))).reshape(-1)
    nb_p = jnp.pad(neigh_ids_B.astype(i32), ((0, pad), (0, 0))).reshape(-1)

    out_pre = _sc_attention(comb, na_p, nb_p, tgt_p,
                            sa, qa, sb, qb, wt, qt)

    y = _out_proj(out_pre, proj_w, proj_b.reshape(1, D))
    return y[:N]
